# baseline (speedup 1.0000x reference)
import functools

import jax
import jax.numpy as jnp
from jax import lax
from jax.experimental import pallas as pl
from jax.experimental.pallas import tpu as pltpu
from jax.experimental.pallas import tpu_sc as plsc

B = 16384
V_CHR, V_STRAND, V_CAS9, V_SOURCE = 25, 2, 8, 16
D_CHR, D_STRAND, D_CAS9, D_SOURCE = 8, 2, 8, 8
BLK = 2048

NC, NS = 2, 16
NW = NC * NS
BPW = B // NW
CHUNK = 128
NCH = BPW // CHUNK


def _sc_gather_body(echr_t, estr_t, ecas_t, esrc_t,
                    ichr_h, istr_h, icas_h, isrc_h,
                    ochr, ostr, ocas, osrc,
                    idx_v, rchr_v, rstr_v, rcas_v, rsrc_v, sem):
    wid = lax.axis_index("s") * NC + lax.axis_index("c")
    base = wid * BPW
    crow = wid * NCH

    pltpu.sync_copy(ichr_h.at[pl.ds(crow, NCH)], idx_v.at[0])
    pltpu.sync_copy(istr_h.at[pl.ds(crow, NCH)], idx_v.at[1])
    pltpu.sync_copy(icas_h.at[pl.ds(crow, NCH)], idx_v.at[2])
    pltpu.sync_copy(isrc_h.at[pl.ds(crow, NCH)], idx_v.at[3])

    copies = []
    for j in range(NCH):
        s = pl.ds(j * CHUNK, CHUNK)
        copies.append(pltpu.async_copy(echr_t.at[idx_v.at[0, j]], rchr_v.at[s], sem))
        copies.append(pltpu.async_copy(estr_t.at[idx_v.at[1, j]], rstr_v.at[s], sem))
        copies.append(pltpu.async_copy(ecas_t.at[idx_v.at[2, j]], rcas_v.at[s], sem))
        copies.append(pltpu.async_copy(esrc_t.at[idx_v.at[3, j]], rsrc_v.at[s], sem))
    for c in copies:
        c.wait()

    pltpu.sync_copy(rchr_v, ochr.at[pl.ds(base, BPW)])
    pltpu.sync_copy(rstr_v, ostr.at[pl.ds(base, BPW)])
    pltpu.sync_copy(rcas_v, ocas.at[pl.ds(base, BPW)])
    pltpu.sync_copy(rsrc_v, osrc.at[pl.ds(base, BPW)])


_sc_gather = functools.partial(
    pl.kernel,
    mesh=plsc.VectorSubcoreMesh(core_axis_name="c", subcore_axis_name="s"),
    compiler_params=pltpu.CompilerParams(use_tc_tiling_on_sc=False),
    out_type=[
        jax.ShapeDtypeStruct((B, D_CHR), jnp.float32),
        jax.ShapeDtypeStruct((B, 8), jnp.float32),
        jax.ShapeDtypeStruct((B, D_CAS9), jnp.float32),
        jax.ShapeDtypeStruct((B, D_SOURCE), jnp.float32),
    ],
    scratch_types=[
        pltpu.VMEM((4, NCH, CHUNK), jnp.int32),
        pltpu.VMEM((BPW, D_CHR), jnp.float32),
        pltpu.VMEM((BPW, 8), jnp.float32),
        pltpu.VMEM((BPW, D_CAS9), jnp.float32),
        pltpu.VMEM((BPW, D_SOURCE), jnp.float32),
        pltpu.SemaphoreType.DMA,
    ],
)(_sc_gather_body)


def _mlp_body(xnum_ref, echr_ref, estr_ref, ecas_ref, esrc_ref,
              w1_ref, b1_ref, w2_ref, b2_ref, out_ref):
    w1 = w1_ref[...]
    f32 = jnp.float32
    h = jnp.dot(xnum_ref[...], w1[0:2, :], preferred_element_type=f32)
    h = h + jnp.dot(echr_ref[...], w1[2:10, :], preferred_element_type=f32)
    h = h + jnp.dot(estr_ref[...], w1[10:18, :], preferred_element_type=f32)
    h = h + jnp.dot(ecas_ref[...], w1[12:20, :], preferred_element_type=f32)
    h = h + jnp.dot(esrc_ref[...], w1[20:28, :], preferred_element_type=f32)
    h = jnp.maximum(h + b1_ref[...], 0.0)
    out_ref[...] = jnp.dot(h, w2_ref[...], preferred_element_type=f32) + b2_ref[...]


def _mlp(x_num, e_chr, e_strand, e_cas9, e_source, W1, b1, W2, b2):
    data = lambda i: (i, 0)
    full = lambda i: (0, 0)
    return pl.pallas_call(
        _mlp_body,
        grid=(B // BLK,),
        in_specs=[
            pl.BlockSpec((BLK, 2), data),
            pl.BlockSpec((BLK, D_CHR), data),
            pl.BlockSpec((BLK, 8), data),
            pl.BlockSpec((BLK, D_CAS9), data),
            pl.BlockSpec((BLK, D_SOURCE), data),
            pl.BlockSpec((28, 64), full),
            pl.BlockSpec((1, 64), full),
            pl.BlockSpec((64, 32), full),
            pl.BlockSpec((1, 32), full),
        ],
        out_specs=pl.BlockSpec((BLK, 32), data),
        out_shape=jax.ShapeDtypeStruct((B, 32), jnp.float32),
    )(x_num, e_chr, e_strand, e_cas9, e_source, W1, b1, W2, b2)


@jax.jit
def _run(x_num, ichr, istr, icas, isrc,
         emb_chr, emb_strand, emb_cas9, emb_source, W1, b1, W2, b2):
    e_chr, e_strand, e_cas9, e_source = _sc_gather(
        emb_chr, emb_strand, emb_cas9, emb_source, ichr, istr, icas, isrc)
    return _mlp(x_num, e_chr, e_strand, e_cas9, e_source, W1, b1, W2, b2)


def kernel(x_num, x_chr, x_strand, x_cas9, x_source,
           emb_chr, emb_strand, emb_cas9, emb_source,
           W1, b1, W2, b2):
    emb_strand = jnp.pad(emb_strand, ((0, 0), (0, 6)))
    ichr = x_chr.astype(jnp.int32).reshape(B // CHUNK, CHUNK)
    istr = x_strand.astype(jnp.int32).reshape(B // CHUNK, CHUNK)
    icas = x_cas9.astype(jnp.int32).reshape(B // CHUNK, CHUNK)
    isrc = x_source.astype(jnp.int32).reshape(B // CHUNK, CHUNK)
    return _run(x_num, ichr, istr, icas, isrc,
                emb_chr, emb_strand, emb_cas9, emb_source,
                W1, b1.reshape(1, 64), W2, b2.reshape(1, 32))


# speedup vs baseline: 2.0196x; 2.0196x over previous
import functools

import jax
import jax.numpy as jnp
from jax import lax
from jax.experimental import pallas as pl
from jax.experimental.pallas import tpu as pltpu
from jax.experimental.pallas import tpu_sc as plsc

B = 16384
V_CHR, V_STRAND, V_CAS9, V_SOURCE = 25, 2, 8, 16
D_CHR, D_STRAND, D_CAS9, D_SOURCE = 8, 2, 8, 8
BLK = 2048

NC, NS = 2, 16
NW = NC * NS
BPW = B // NW
CHUNK = 128
NCH = BPW // CHUNK


def _sc_gather_body(echr_t, estr_t, ecas_t, esrc_t, idx_h,
                    ochr, ostr, ocas, osrc,
                    idx_v, tchr_v, tstr_v, tcas_v, tsrc_v,
                    rchr_v, rstr_v, rcas_v, rsrc_v, sem):
    wid = lax.axis_index("s") * NC + lax.axis_index("c")
    base = wid * BPW
    crow = wid * NCH

    @pl.when(lax.axis_index("s") == 0)
    def _stage_tables():
        pltpu.sync_copy(echr_t, tchr_v)
        pltpu.sync_copy(estr_t, tstr_v)
        pltpu.sync_copy(ecas_t, tcas_v)
        pltpu.sync_copy(esrc_t, tsrc_v)

    pltpu.sync_copy(idx_h.at[pl.ds(crow, NCH)], idx_v)
    plsc.subcore_barrier()

    copies = []
    for j in range(NCH):
        s = pl.ds(j * CHUNK, CHUNK)
        copies.append(pltpu.async_copy(tchr_v.at[idx_v.at[j, 0]], rchr_v.at[s], sem))
        copies.append(pltpu.async_copy(tstr_v.at[idx_v.at[j, 1]], rstr_v.at[s], sem))
        copies.append(pltpu.async_copy(tcas_v.at[idx_v.at[j, 2]], rcas_v.at[s], sem))
        copies.append(pltpu.async_copy(tsrc_v.at[idx_v.at[j, 3]], rsrc_v.at[s], sem))
    for c in copies:
        c.wait()

    pltpu.sync_copy(rchr_v, ochr.at[pl.ds(base, BPW)])
    pltpu.sync_copy(rstr_v, ostr.at[pl.ds(base, BPW)])
    pltpu.sync_copy(rcas_v, ocas.at[pl.ds(base, BPW)])
    pltpu.sync_copy(rsrc_v, osrc.at[pl.ds(base, BPW)])


_sc_gather = functools.partial(
    pl.kernel,
    mesh=plsc.VectorSubcoreMesh(core_axis_name="c", subcore_axis_name="s"),
    compiler_params=pltpu.CompilerParams(use_tc_tiling_on_sc=False),
    out_type=[
        jax.ShapeDtypeStruct((B, D_CHR), jnp.float32),
        jax.ShapeDtypeStruct((B, 8), jnp.float32),
        jax.ShapeDtypeStruct((B, D_CAS9), jnp.float32),
        jax.ShapeDtypeStruct((B, D_SOURCE), jnp.float32),
    ],
    scratch_types=[
        pltpu.VMEM((NCH, 4, CHUNK), jnp.int32),
        pltpu.VMEM_SHARED((V_CHR, D_CHR), jnp.float32),
        pltpu.VMEM_SHARED((V_STRAND, 8), jnp.float32),
        pltpu.VMEM_SHARED((V_CAS9, D_CAS9), jnp.float32),
        pltpu.VMEM_SHARED((V_SOURCE, D_SOURCE), jnp.float32),
        pltpu.VMEM((BPW, D_CHR), jnp.float32),
        pltpu.VMEM((BPW, 8), jnp.float32),
        pltpu.VMEM((BPW, D_CAS9), jnp.float32),
        pltpu.VMEM((BPW, D_SOURCE), jnp.float32),
        pltpu.SemaphoreType.DMA,
    ],
)(_sc_gather_body)


def _mlp_body(xnum_ref, echr_ref, estr_ref, ecas_ref, esrc_ref,
              w1_ref, b1_ref, w2_ref, b2_ref, out_ref):
    w1 = w1_ref[...]
    f32 = jnp.float32
    h = jnp.dot(xnum_ref[...], w1[0:2, :], preferred_element_type=f32)
    h = h + jnp.dot(echr_ref[...], w1[2:10, :], preferred_element_type=f32)
    h = h + jnp.dot(estr_ref[...], w1[10:18, :], preferred_element_type=f32)
    h = h + jnp.dot(ecas_ref[...], w1[12:20, :], preferred_element_type=f32)
    h = h + jnp.dot(esrc_ref[...], w1[20:28, :], preferred_element_type=f32)
    h = jnp.maximum(h + b1_ref[...], 0.0)
    out_ref[...] = jnp.dot(h, w2_ref[...], preferred_element_type=f32) + b2_ref[...]


def _mlp(x_num, e_chr, e_strand, e_cas9, e_source, W1, b1, W2, b2):
    data = lambda i: (i, 0)
    full = lambda i: (0, 0)
    return pl.pallas_call(
        _mlp_body,
        grid=(B // BLK,),
        in_specs=[
            pl.BlockSpec((BLK, 2), data),
            pl.BlockSpec((BLK, D_CHR), data),
            pl.BlockSpec((BLK, 8), data),
            pl.BlockSpec((BLK, D_CAS9), data),
            pl.BlockSpec((BLK, D_SOURCE), data),
            pl.BlockSpec((28, 64), full),
            pl.BlockSpec((1, 64), full),
            pl.BlockSpec((64, 32), full),
            pl.BlockSpec((1, 32), full),
        ],
        out_specs=pl.BlockSpec((BLK, 32), data),
        out_shape=jax.ShapeDtypeStruct((B, 32), jnp.float32),
    )(x_num, e_chr, e_strand, e_cas9, e_source, W1, b1, W2, b2)


@jax.jit
def _run(x_num, idx_all,
         emb_chr, emb_strand, emb_cas9, emb_source, W1, b1, W2, b2):
    e_chr, e_strand, e_cas9, e_source = _sc_gather(
        emb_chr, emb_strand, emb_cas9, emb_source, idx_all)
    return _mlp(x_num, e_chr, e_strand, e_cas9, e_source, W1, b1, W2, b2)


def kernel(x_num, x_chr, x_strand, x_cas9, x_source,
           emb_chr, emb_strand, emb_cas9, emb_source,
           W1, b1, W2, b2):
    emb_strand = jnp.pad(emb_strand, ((0, 0), (0, 6)))
    idx_all = jnp.stack(
        [x_chr.astype(jnp.int32).reshape(B // CHUNK, CHUNK),
         x_strand.astype(jnp.int32).reshape(B // CHUNK, CHUNK),
         x_cas9.astype(jnp.int32).reshape(B // CHUNK, CHUNK),
         x_source.astype(jnp.int32).reshape(B // CHUNK, CHUNK)], axis=1)
    return _run(x_num, idx_all,
                emb_chr, emb_strand, emb_cas9, emb_source,
                W1, b1.reshape(1, 64), W2, b2.reshape(1, 32))


# speedup vs baseline: 3.0082x; 1.4895x over previous
import functools

import jax
import jax.numpy as jnp
from jax import lax
from jax.experimental import pallas as pl
from jax.experimental.pallas import tpu as pltpu
from jax.experimental.pallas import tpu_sc as plsc

B = 16384
V_CHR, V_STRAND, V_CAS9, V_SOURCE = 25, 2, 8, 16
VC = V_CHR * V_STRAND * V_CAS9 * V_SOURCE
HID = 64
BLK = 2048

NC, NS = 2, 16
NW = NC * NS
BPW = B // NW
CHUNK = 128
NCH = BPW // CHUNK
LANES = 16


def _prep_body(echr_ref, estr_ref, ecas_ref, esrc_ref, w1_ref, b1_ref, out_ref):
    w1 = w1_ref[...]
    f32 = jnp.float32
    f_chr = jnp.dot(echr_ref[...], w1[2:10, :], preferred_element_type=f32)
    f_str = jnp.dot(estr_ref[...], w1[10:12, :], preferred_element_type=f32)
    f_cas = jnp.dot(ecas_ref[...], w1[12:20, :], preferred_element_type=f32)
    f_src = jnp.dot(esrc_ref[...], w1[20:28, :], preferred_element_type=f32)

    r = jax.lax.broadcasted_iota(jnp.int32, (VC, 1), 0)
    c = r >> 8
    s = (r >> 7) & 1
    k = (r >> 4) & 7
    src = r & 15

    def oh(idx_col, V):
        iota = jax.lax.broadcasted_iota(jnp.int32, (1, V), 1)
        return (idx_col == iota).astype(f32)

    t = jnp.dot(oh(c, V_CHR), f_chr, preferred_element_type=f32)
    t = t + jnp.dot(oh(s, V_STRAND), f_str, preferred_element_type=f32)
    t = t + jnp.dot(oh(k, V_CAS9), f_cas, preferred_element_type=f32)
    t = t + jnp.dot(oh(src, V_SOURCE), f_src, preferred_element_type=f32)
    t = t + b1_ref[...]
    out_ref[...] = jnp.pad(t, ((0, 0), (0, 128 - HID)))


def _prep(emb_chr, emb_strand, emb_cas9, emb_source, W1, b1):
    full = lambda: (0, 0)
    return pl.pallas_call(
        _prep_body,
        in_specs=[
            pl.BlockSpec((V_CHR, 8), full),
            pl.BlockSpec((V_STRAND, 2), full),
            pl.BlockSpec((V_CAS9, 8), full),
            pl.BlockSpec((V_SOURCE, 8), full),
            pl.BlockSpec((28, 64), full),
            pl.BlockSpec((1, 64), full),
        ],
        out_specs=pl.BlockSpec((VC, 128), full),
        out_shape=jax.ShapeDtypeStruct((VC, 128), jnp.float32),
    )(emb_chr, emb_strand, emb_cas9, emb_source, W1, b1)


def _sc_gather_body(table_h, ichr_h, istr_h, icas_h, isrc_h,
                    z_out,
                    table_sp, ichr_v, istr_v, icas_v, isrc_v, cidx_v, z_v, sem):
    wid = lax.axis_index("s") * NC + lax.axis_index("c")
    base = wid * BPW

    @pl.when(lax.axis_index("s") == 0)
    def _stage_table():
        pltpu.sync_copy(table_h, table_sp)

    pltpu.sync_copy(ichr_h.at[pl.ds(base, BPW)], ichr_v)
    pltpu.sync_copy(istr_h.at[pl.ds(base, BPW)], istr_v)
    pltpu.sync_copy(icas_h.at[pl.ds(base, BPW)], icas_v)
    pltpu.sync_copy(isrc_h.at[pl.ds(base, BPW)], isrc_v)
    for g in range(BPW // LANES):
        sl = pl.ds(g * LANES, LANES)
        cidx_v[sl] = ((ichr_v[sl] * 2 + istr_v[sl]) * 8 + icas_v[sl]) * 16 + isrc_v[sl]

    plsc.subcore_barrier()

    copies = []
    for j in range(NCH):
        s = pl.ds(j * CHUNK, CHUNK)
        copies.append(pltpu.async_copy(table_sp.at[cidx_v.at[s]], z_v.at[s], sem))
    for c in copies:
        c.wait()

    pltpu.sync_copy(z_v, z_out.at[pl.ds(base, BPW)])


_sc_gather = functools.partial(
    pl.kernel,
    mesh=plsc.VectorSubcoreMesh(core_axis_name="c", subcore_axis_name="s"),
    compiler_params=pltpu.CompilerParams(use_tc_tiling_on_sc=False),
    out_type=jax.ShapeDtypeStruct((B, 128), jnp.float32),
    scratch_types=[
        pltpu.VMEM_SHARED((VC, 128), jnp.float32),
        pltpu.VMEM((BPW,), jnp.int32),
        pltpu.VMEM((BPW,), jnp.int32),
        pltpu.VMEM((BPW,), jnp.int32),
        pltpu.VMEM((BPW,), jnp.int32),
        pltpu.VMEM((BPW,), jnp.int32),
        pltpu.VMEM((BPW, 128), jnp.float32),
        pltpu.SemaphoreType.DMA,
    ],
)(_sc_gather_body)


def _fin_body(z_ref, xnum_ref, w1_ref, w2_ref, b2_ref, out_ref):
    f32 = jnp.float32
    z = z_ref[...]
    h = z[:, 0:HID] + jnp.dot(xnum_ref[...], w1_ref[0:2, :],
                              preferred_element_type=f32)
    h = jnp.maximum(h, 0.0)
    out_ref[...] = jnp.dot(h, w2_ref[...], preferred_element_type=f32) + b2_ref[...]


def _fin(z, x_num, W1, W2, b2):
    data = lambda i: (i, 0)
    full = lambda i: (0, 0)
    return pl.pallas_call(
        _fin_body,
        grid=(B // BLK,),
        in_specs=[
            pl.BlockSpec((BLK, 128), data),
            pl.BlockSpec((BLK, 2), data),
            pl.BlockSpec((28, 64), full),
            pl.BlockSpec((64, 32), full),
            pl.BlockSpec((1, 32), full),
        ],
        out_specs=pl.BlockSpec((BLK, 32), data),
        out_shape=jax.ShapeDtypeStruct((B, 32), jnp.float32),
    )(z, x_num, W1, W2, b2)


@jax.jit
def _run(x_num, ichr, istr, icas, isrc,
         emb_chr, emb_strand, emb_cas9, emb_source, W1, b1, W2, b2):
    table = _prep(emb_chr, emb_strand, emb_cas9, emb_source, W1, b1)
    z = _sc_gather(table, ichr, istr, icas, isrc)
    return _fin(z, x_num, W1, W2, b2)


def kernel(x_num, x_chr, x_strand, x_cas9, x_source,
           emb_chr, emb_strand, emb_cas9, emb_source,
           W1, b1, W2, b2):
    return _run(x_num, x_chr.astype(jnp.int32), x_strand.astype(jnp.int32),
                x_cas9.astype(jnp.int32), x_source.astype(jnp.int32),
                emb_chr, emb_strand, emb_cas9, emb_source,
                W1, b1.reshape(1, 64), W2, b2.reshape(1, 32))


# speedup vs baseline: 3.6244x; 1.2048x over previous
import functools

import jax
import jax.numpy as jnp
from jax import lax
from jax.experimental import pallas as pl
from jax.experimental.pallas import tpu as pltpu
from jax.experimental.pallas import tpu_sc as plsc

B = 16384
V_CHR, V_STRAND, V_CAS9, V_SOURCE = 25, 2, 8, 16
VC = V_CHR * V_STRAND * V_CAS9 * V_SOURCE
HID = 64
BLK = 2048

NC, NS = 2, 16
NW = NC * NS
BPW = B // NW
CHUNK = 128
NCH = BPW // CHUNK
LANES = 16


def _prep_body(echrT_ref, estr_ref, ecas_ref, esrcT_ref, w1_ref, b1_ref, out_ref):
    w1 = w1_ref[...]
    f32 = jnp.float32
    e_chr = jnp.transpose(echrT_ref[...])
    e_src = jnp.transpose(esrcT_ref[...])
    f_chr = jnp.dot(e_chr, w1[2:10, :], preferred_element_type=f32)
    f_str = jnp.dot(estr_ref[...], w1[10:12, :], preferred_element_type=f32)
    f_cas = jnp.dot(ecas_ref[...], w1[12:20, :], preferred_element_type=f32)
    f_src = jnp.dot(e_src, w1[20:28, :], preferred_element_type=f32)

    r = jax.lax.broadcasted_iota(jnp.int32, (VC, 1), 0)
    c = r >> 8
    s = (r >> 7) & 1
    k = (r >> 4) & 7
    src = r & 15

    def oh(idx_col, V):
        iota = jax.lax.broadcasted_iota(jnp.int32, (1, V), 1)
        return (idx_col == iota).astype(f32)

    t = jnp.dot(oh(c, V_CHR), f_chr, preferred_element_type=f32)
    t = t + jnp.dot(oh(s, V_STRAND), f_str, preferred_element_type=f32)
    t = t + jnp.dot(oh(k, V_CAS9), f_cas, preferred_element_type=f32)
    t = t + jnp.dot(oh(src, V_SOURCE), f_src, preferred_element_type=f32)
    t = t + b1_ref[...]
    out_ref[...] = jnp.pad(t, ((0, 0), (0, 128 - HID)))


def _prep(emb_chrT, emb_strand, emb_cas9, emb_sourceT, W1, b1):
    full = lambda: (0, 0)
    return pl.pallas_call(
        _prep_body,
        in_specs=[
            pl.BlockSpec((8, V_CHR), full),
            pl.BlockSpec((V_STRAND, 2), full),
            pl.BlockSpec((V_CAS9, 8), full),
            pl.BlockSpec((8, V_SOURCE), full),
            pl.BlockSpec((28, 64), full),
            pl.BlockSpec((1, 64), full),
        ],
        out_specs=pl.BlockSpec((VC, 128), full),
        out_shape=jax.ShapeDtypeStruct((VC, 128), jnp.float32),
    )(emb_chrT, emb_strand, emb_cas9, emb_sourceT, W1, b1)


def _sc_gather_body(table_h, ichr_h, istr_h, icas_h, isrc_h, xT_h,
                    z_out,
                    table_sp, ichr_v, istr_v, icas_v, isrc_v, cidx_v,
                    x0_v, x1_v, z_v, sem):
    wid = lax.axis_index("s") * NC + lax.axis_index("c")
    base = wid * BPW

    @pl.when(lax.axis_index("s") == 0)
    def _stage_table():
        pltpu.sync_copy(table_h, table_sp)

    pltpu.sync_copy(ichr_h.at[pl.ds(base, BPW)], ichr_v)
    pltpu.sync_copy(istr_h.at[pl.ds(base, BPW)], istr_v)
    pltpu.sync_copy(icas_h.at[pl.ds(base, BPW)], icas_v)
    pltpu.sync_copy(isrc_h.at[pl.ds(base, BPW)], isrc_v)
    pltpu.sync_copy(xT_h.at[0, pl.ds(base, BPW)], x0_v)
    pltpu.sync_copy(xT_h.at[1, pl.ds(base, BPW)], x1_v)
    for g in range(BPW // LANES):
        sl = pl.ds(g * LANES, LANES)
        cidx_v[sl] = ((ichr_v[sl] * 2 + istr_v[sl]) * 8 + icas_v[sl]) * 16 + isrc_v[sl]

    plsc.subcore_barrier()

    copies = []
    for j in range(NCH):
        s = pl.ds(j * CHUNK, CHUNK)
        copies.append(pltpu.async_copy(table_sp.at[cidx_v.at[s]], z_v.at[s], sem))
    for c in copies:
        c.wait()

    rows16 = lax.iota(jnp.int32, LANES)
    for g in range(BPW // LANES):
        rows = rows16 + (g * LANES)
        sl = pl.ds(g * LANES, LANES)
        plsc.store_scatter(z_v, [rows, jnp.full((LANES,), 64, jnp.int32)], x0_v[sl])
        plsc.store_scatter(z_v, [rows, jnp.full((LANES,), 65, jnp.int32)], x1_v[sl])

    pltpu.sync_copy(z_v, z_out.at[pl.ds(base, BPW)])


_sc_gather = functools.partial(
    pl.kernel,
    mesh=plsc.VectorSubcoreMesh(core_axis_name="c", subcore_axis_name="s"),
    compiler_params=pltpu.CompilerParams(use_tc_tiling_on_sc=False,
                                         needs_layout_passes=False),
    out_type=jax.ShapeDtypeStruct((B, 128), jnp.float32),
    scratch_types=[
        pltpu.VMEM_SHARED((VC, 128), jnp.float32),
        pltpu.VMEM((BPW,), jnp.int32),
        pltpu.VMEM((BPW,), jnp.int32),
        pltpu.VMEM((BPW,), jnp.int32),
        pltpu.VMEM((BPW,), jnp.int32),
        pltpu.VMEM((BPW,), jnp.int32),
        pltpu.VMEM((BPW,), jnp.float32),
        pltpu.VMEM((BPW,), jnp.float32),
        pltpu.VMEM((BPW, 128), jnp.float32),
        pltpu.SemaphoreType.DMA,
    ],
)(_sc_gather_body)


def _fin_body(z_ref, w1_ref, w2T_ref, b2T_ref, outT_ref):
    f32 = jnp.float32
    i32 = jnp.int32
    eye = (jax.lax.broadcasted_iota(i32, (HID, HID), 0)
           == jax.lax.broadcasted_iota(i32, (HID, HID), 1)).astype(f32)
    m = jnp.concatenate([eye, w1_ref[0:2, :], jnp.zeros((62, HID), f32)], axis=0)
    h = jnp.maximum(jnp.dot(z_ref[...], m, preferred_element_type=f32), 0.0)
    w2 = jnp.transpose(w2T_ref[...])
    out = jnp.dot(h, w2, preferred_element_type=f32)
    outT_ref[...] = jnp.transpose(out) + b2T_ref[...]


def _fin(z, W1, W2T, b2T):
    data = lambda i: (i, 0)
    dataT = lambda i: (0, i)
    full = lambda i: (0, 0)
    return pl.pallas_call(
        _fin_body,
        grid=(B // BLK,),
        in_specs=[
            pl.BlockSpec((BLK, 128), data),
            pl.BlockSpec((28, 64), full),
            pl.BlockSpec((32, 64), full),
            pl.BlockSpec((32, 1), full),
        ],
        out_specs=pl.BlockSpec((32, BLK), dataT),
        out_shape=jax.ShapeDtypeStruct((32, B), jnp.float32),
    )(z, W1, W2T, b2T)


@jax.jit
def _run(xT, ichr, istr, icas, isrc,
         emb_chrT, emb_strand, emb_cas9, emb_sourceT, W1, b1, W2T, b2T):
    table = _prep(emb_chrT, emb_strand, emb_cas9, emb_sourceT, W1, b1)
    z = _sc_gather(table, ichr, istr, icas, isrc, xT)
    return jnp.transpose(_fin(z, W1, W2T, b2T))


def kernel(x_num, x_chr, x_strand, x_cas9, x_source,
           emb_chr, emb_strand, emb_cas9, emb_source,
           W1, b1, W2, b2):
    return _run(jnp.transpose(x_num),
                x_chr.astype(jnp.int32), x_strand.astype(jnp.int32),
                x_cas9.astype(jnp.int32), x_source.astype(jnp.int32),
                jnp.transpose(emb_chr), emb_strand, emb_cas9,
                jnp.transpose(emb_source),
                W1, b1.reshape(1, 64), jnp.transpose(W2), b2.reshape(32, 1))


# speedup vs baseline: 4.0255x; 1.1107x over previous
import functools

import jax
import jax.numpy as jnp
from jax import lax
from jax.experimental import pallas as pl
from jax.experimental.pallas import tpu as pltpu
from jax.experimental.pallas import tpu_sc as plsc

B = 16384
V_CHR, V_STRAND, V_CAS9, V_SOURCE = 25, 2, 8, 16
VC = V_CHR * V_STRAND * V_CAS9 * V_SOURCE
HID = 64
BLK = 2048

NC, NS = 2, 16
NW = NC * NS
BPW = B // NW
CHUNK = 128
NCH = BPW // CHUNK
LANES = 16


def _prep_body(echrT_ref, estr_ref, ecas_ref, esrcT_ref, w1_ref, b1_ref,
               ichr_ref, istr_ref, icas_ref, isrc_ref, out_ref, cidx_ref):
    w1 = w1_ref[...]
    f32 = jnp.float32
    e_chr = jnp.transpose(echrT_ref[...])
    e_src = jnp.transpose(esrcT_ref[...])
    f_chr = jnp.dot(e_chr, w1[2:10, :], preferred_element_type=f32)
    f_str = jnp.dot(estr_ref[...], w1[10:12, :], preferred_element_type=f32)
    f_cas = jnp.dot(ecas_ref[...], w1[12:20, :], preferred_element_type=f32)
    f_src = jnp.dot(e_src, w1[20:28, :], preferred_element_type=f32)

    r = jax.lax.broadcasted_iota(jnp.int32, (VC, 1), 0)
    c = r >> 8
    s = (r >> 7) & 1
    k = (r >> 4) & 7
    src = r & 15

    def oh(idx_col, V):
        iota = jax.lax.broadcasted_iota(jnp.int32, (1, V), 1)
        return (idx_col == iota).astype(f32)

    t = jnp.dot(oh(c, V_CHR), f_chr, preferred_element_type=f32)
    t = t + jnp.dot(oh(s, V_STRAND), f_str, preferred_element_type=f32)
    t = t + jnp.dot(oh(k, V_CAS9), f_cas, preferred_element_type=f32)
    t = t + jnp.dot(oh(src, V_SOURCE), f_src, preferred_element_type=f32)
    t = t + b1_ref[...]
    out_ref[...] = jnp.pad(t, ((0, 0), (0, 128 - HID)))

    cidx_ref[...] = ((ichr_ref[...] * 2 + istr_ref[...]) * 8
                     + icas_ref[...]) * 16 + isrc_ref[...]


def _prep(emb_chrT, emb_strand, emb_cas9, emb_sourceT, W1, b1,
          ichr2, istr2, icas2, isrc2):
    full = lambda: (0, 0)
    return pl.pallas_call(
        _prep_body,
        in_specs=[
            pl.BlockSpec((8, V_CHR), full),
            pl.BlockSpec((V_STRAND, 2), full),
            pl.BlockSpec((V_CAS9, 8), full),
            pl.BlockSpec((8, V_SOURCE), full),
            pl.BlockSpec((28, 64), full),
            pl.BlockSpec((1, 64), full),
            pl.BlockSpec((B // 128, 128), full),
            pl.BlockSpec((B // 128, 128), full),
            pl.BlockSpec((B // 128, 128), full),
            pl.BlockSpec((B // 128, 128), full),
        ],
        out_specs=[pl.BlockSpec((VC, 128), full),
                   pl.BlockSpec((B // 128, 128), full)],
        out_shape=[jax.ShapeDtypeStruct((VC, 128), jnp.float32),
                   jax.ShapeDtypeStruct((B // 128, 128), jnp.int32)],
    )(emb_chrT, emb_strand, emb_cas9, emb_sourceT, W1, b1,
      ichr2, istr2, icas2, isrc2)


def _sc_gather_body(table_h, cidx_h, xT_h,
                    z_out,
                    table_sp, cidx_v, x0_v, x1_v, z_v, sem, osem):
    wid = lax.axis_index("s") * NC + lax.axis_index("c")
    base = wid * BPW
    crow = wid * NCH

    @pl.when(lax.axis_index("s") == 0)
    def _stage_table():
        pltpu.sync_copy(table_h, table_sp)

    stage = [pltpu.async_copy(cidx_h.at[pl.ds(crow, NCH)], cidx_v, sem),
             pltpu.async_copy(xT_h.at[0, pl.ds(base, BPW)], x0_v, sem),
             pltpu.async_copy(xT_h.at[1, pl.ds(base, BPW)], x1_v, sem)]
    for c in stage:
        c.wait()

    plsc.subcore_barrier()

    copies = [pltpu.async_copy(table_sp.at[cidx_v.at[j]],
                               z_v.at[pl.ds(j * CHUNK, CHUNK)], sem)
              for j in range(NCH)]

    rows16 = lax.iota(jnp.int32, LANES)
    c64 = jnp.full((LANES,), 64, jnp.int32)
    c65 = jnp.full((LANES,), 65, jnp.int32)
    out_copies = []
    for j in range(NCH):
        copies[j].wait()
        for g in range(CHUNK // LANES):
            rows = rows16 + (j * CHUNK + g * LANES)
            sl = pl.ds(j * CHUNK + g * LANES, LANES)
            plsc.store_scatter(z_v, [rows, c64], x0_v[sl])
            plsc.store_scatter(z_v, [rows, c65], x1_v[sl])
        out_copies.append(
            pltpu.async_copy(z_v.at[pl.ds(j * CHUNK, CHUNK)],
                             z_out.at[pl.ds(base + j * CHUNK, CHUNK)], osem))
    for c in out_copies:
        c.wait()


_sc_gather = functools.partial(
    pl.kernel,
    mesh=plsc.VectorSubcoreMesh(core_axis_name="c", subcore_axis_name="s"),
    compiler_params=pltpu.CompilerParams(use_tc_tiling_on_sc=False,
                                         needs_layout_passes=False),
    out_type=jax.ShapeDtypeStruct((B, 128), jnp.float32),
    scratch_types=[
        pltpu.VMEM_SHARED((VC, 128), jnp.float32),
        pltpu.VMEM((NCH, CHUNK), jnp.int32),
        pltpu.VMEM((BPW,), jnp.float32),
        pltpu.VMEM((BPW,), jnp.float32),
        pltpu.VMEM((BPW, 128), jnp.float32),
        pltpu.SemaphoreType.DMA,
        pltpu.SemaphoreType.DMA,
    ],
)(_sc_gather_body)


def _fin_body(z_ref, w1_ref, w2T_ref, b2T_ref, outT_ref):
    f32 = jnp.float32
    i32 = jnp.int32
    eye = (jax.lax.broadcasted_iota(i32, (HID, HID), 0)
           == jax.lax.broadcasted_iota(i32, (HID, HID), 1)).astype(f32)
    m = jnp.concatenate([eye, w1_ref[0:2, :], jnp.zeros((62, HID), f32)], axis=0)
    h = jnp.maximum(jnp.dot(z_ref[...], m, preferred_element_type=f32), 0.0)
    w2 = jnp.transpose(w2T_ref[...])
    out = jnp.dot(h, w2, preferred_element_type=f32)
    outT_ref[...] = jnp.transpose(out) + b2T_ref[...]


def _fin(z, W1, W2T, b2T):
    data = lambda i: (i, 0)
    dataT = lambda i: (0, i)
    full = lambda i: (0, 0)
    return pl.pallas_call(
        _fin_body,
        grid=(B // BLK,),
        in_specs=[
            pl.BlockSpec((BLK, 128), data),
            pl.BlockSpec((28, 64), full),
            pl.BlockSpec((32, 64), full),
            pl.BlockSpec((32, 1), full),
        ],
        out_specs=pl.BlockSpec((32, BLK), dataT),
        out_shape=jax.ShapeDtypeStruct((32, B), jnp.float32),
    )(z, W1, W2T, b2T)


@jax.jit
def _run(xT, ichr2, istr2, icas2, isrc2,
         emb_chrT, emb_strand, emb_cas9, emb_sourceT, W1, b1, W2T, b2T):
    table, cidx = _prep(emb_chrT, emb_strand, emb_cas9, emb_sourceT, W1, b1,
                        ichr2, istr2, icas2, isrc2)
    z = _sc_gather(table, cidx, xT)
    return jnp.transpose(_fin(z, W1, W2T, b2T))


def kernel(x_num, x_chr, x_strand, x_cas9, x_source,
           emb_chr, emb_strand, emb_cas9, emb_source,
           W1, b1, W2, b2):
    as2d = lambda a: a.astype(jnp.int32).reshape(B // 128, 128)
    return _run(jnp.transpose(x_num),
                as2d(x_chr), as2d(x_strand), as2d(x_cas9), as2d(x_source),
                jnp.transpose(emb_chr), emb_strand, emb_cas9,
                jnp.transpose(emb_source),
                W1, b1.reshape(1, 64), jnp.transpose(W2), b2.reshape(32, 1))
